# baseline (speedup 1.0000x reference)
import functools

import jax
import jax.numpy as jnp
from jax import lax
from jax.experimental import pallas as pl
from jax.experimental.pallas import tpu as pltpu
from jax.experimental.pallas import tpu_sc as plsc

VOCAB = 100000
CONTEXT = 5
EMB = 64
BATCH = 1024
HIDDEN = 128
NLOOK = BATCH * 2 * CONTEXT
IN_FEAT = 2 * CONTEXT * EMB


_NC = 2
_NS = 16
_NW = _NC * _NS
_BPW = NLOOK // _NW


@functools.cache
def _sc_gather_fn():
    mesh = plsc.VectorSubcoreMesh(core_axis_name="c", subcore_axis_name="s")

    @functools.partial(
        pl.kernel,
        mesh=mesh,
        out_type=jax.ShapeDtypeStruct((NLOOK, EMB), jnp.float32),
        scratch_types=[
            pltpu.VMEM((_BPW,), jnp.int32),
            pltpu.VMEM((_BPW, EMB), jnp.float32),
            pltpu.SemaphoreType.DMA,
        ],
        compiler_params=pltpu.CompilerParams(use_tc_tiling_on_sc=False),
    )
    def _sc_gather(idx_hbm, table_hbm, out_hbm, idx_v, rows_v, sem):
        wid = lax.axis_index("s") * _NC + lax.axis_index("c")
        base = wid * _BPW
        pltpu.sync_copy(idx_hbm.at[pl.ds(base, _BPW)], idx_v)
        pltpu.async_copy(table_hbm.at[idx_v], rows_v, sem).wait()
        pltpu.sync_copy(rows_v, out_hbm.at[pl.ds(base, _BPW)])

    return _sc_gather


BS = 32
NB = BATCH // BS


def _hidden_body(e_ref, w1_ref, b1_ref, h_ref):
    hh = jnp.dot(e_ref[...], w1_ref[...], preferred_element_type=jnp.float32)
    h_ref[...] = jnp.maximum(hh + b1_ref[...], 0.0).astype(jnp.bfloat16)


def _fused_body(h_ref, w2_ref, b2_ref, out_ref):
    i = pl.program_id(0)
    hs = h_ref[pl.ds(i * BS, BS), :]
    out_ref[...] = (
        jnp.dot(hs, w2_ref[...], preferred_element_type=jnp.float32)
        + b2_ref[...]
    )
    x = out_ref[...]
    m = jnp.max(x, axis=1, keepdims=True)
    s = jnp.sum(jnp.exp(x - m), axis=1, keepdims=True)
    out_ref[...] = x - (m + jnp.log(s))


def _mlp_logsoftmax(e, W1, b1, W2, b2):
    h = pl.pallas_call(
        _hidden_body,
        out_shape=jax.ShapeDtypeStruct((BATCH, HIDDEN), jnp.bfloat16),
    )(e, W1, b1)

    return pl.pallas_call(
        _fused_body,
        grid=(NB,),
        in_specs=[
            pl.BlockSpec((BATCH, HIDDEN), lambda i: (0, 0)),
            pl.BlockSpec((HIDDEN, VOCAB), lambda i: (0, 0)),
            pl.BlockSpec((1, VOCAB), lambda i: (0, 0)),
        ],
        out_specs=pl.BlockSpec((BS, VOCAB), lambda i: (i, 0)),
        out_shape=jax.ShapeDtypeStruct((BATCH, VOCAB), jnp.float32),
        compiler_params=pltpu.CompilerParams(
            vmem_limit_bytes=127 * 1024 * 1024,
            dimension_semantics=("parallel",),
        ),
    )(h, W2, b2)


def kernel(inputs, embeds, W1, b1, W2, b2):
    idx = inputs.reshape(-1).astype(jnp.int32)
    gathered = _sc_gather_fn()(idx, embeds)
    e = gathered.reshape(BATCH, IN_FEAT)
    return _mlp_logsoftmax(
        e, W1, b1.reshape(1, HIDDEN), W2.astype(jnp.bfloat16), b2.reshape(1, VOCAB)
    )
